# baseline (speedup 1.0000x reference)
import jax
import jax.numpy as jnp
import numpy as np
from jax import lax
from jax.experimental import pallas as pl
from jax.experimental.pallas import tpu as pltpu
from jax.experimental.pallas import tpu_sc as plsc

_B = 128
_D = 2048
_A = 1000
_NS = 8
_KBLK = 2 * _D // _NS
_NA = 8
_ABLK = _A // _NA
_LN2 = 0.6931471805599453
_SQRT2 = 1.4142135623730951


def _threefry2x32_np(k0, k1, x0, x1):
    ks0 = np.uint32(k0)
    ks1 = np.uint32(k1)
    ks2 = np.uint32(ks0 ^ ks1 ^ np.uint32(0x1BD11BDA))
    ks = [ks0, ks1, ks2]
    rot = [[13, 15, 26, 6], [17, 29, 16, 24]]
    x0 = x0 + ks0
    x1 = x1 + ks1
    for r in range(5):
        for ri in rot[r % 2]:
            x0 = x0 + x1
            x1 = (x1 << np.uint32(ri)) | (x1 >> np.uint32(32 - ri))
            x1 = x1 ^ x0
        x0 = x0 + ks[(r + 1) % 3]
        x1 = x1 + ks[(r + 2) % 3] + np.uint32(r + 1)
    return x0, x1


def _gumbel_const(shape, seed):
    n = int(np.prod(shape))
    idx = np.arange(n, dtype=np.uint64)
    c_hi = (idx >> np.uint64(32)).astype(np.uint32)
    c_lo = (idx & np.uint64(0xFFFFFFFF)).astype(np.uint32)
    k0 = np.uint32(seed >> 32)
    k1 = np.uint32(seed & 0xFFFFFFFF)
    with np.errstate(over="ignore"):
        r0, r1 = _threefry2x32_np(k0, k1, c_hi, c_lo)
    bits = r0 ^ r1
    fb = (bits >> np.uint32(9)) | np.uint32(0x3F800000)
    u = fb.view(np.float32) - np.float32(1.0)
    tiny = np.float32(np.finfo(np.float32).tiny)
    u = u * (np.float32(1.0) - tiny) + tiny
    u = np.maximum(tiny, u)
    return (-np.log(-np.log(u))).astype(np.float32).reshape(shape)


_G = _gumbel_const((_B, _A), 42)


def _tc_body(x_ref, b_ref, g_ref, w_hbm, stats_ref, wbuf, sems):
    cps = []
    for i in range(_NS):
        cp = pltpu.make_async_copy(
            w_hbm.at[pl.ds(i * _KBLK, _KBLK), :], wbuf.at[i], sems.at[i])
        cp.start()
        cps.append(cp)

    y = None
    for i in range(_NS):
        cps[i].wait()
        xc0 = (i * _KBLK) % _D
        part = jnp.dot(x_ref[:, xc0:xc0 + _KBLK], wbuf[i],
                       preferred_element_type=jnp.float32)
        y = part if y is None else y + part

    y = y + b_ref[...]
    z = y + g_ref[...]
    for c in range(_NA):
        yc = y[:, c * _ABLK:(c + 1) * _ABLK]
        zc = z[:, c * _ABLK:(c + 1) * _ABLK]
        m = jnp.max(yc, axis=1, keepdims=True)
        e = jnp.exp(yc - m)
        s = jnp.sum(e, axis=1, keepdims=True)
        t = jnp.sum(yc * e, axis=1, keepdims=True)
        bv = jnp.max(zc, axis=1, keepdims=True)
        cols = c * _ABLK + jax.lax.broadcasted_iota(
            jnp.int32, (_B, _ABLK), 1)
        bi = jnp.min(jnp.where(zc == bv, cols, jnp.int32(2**30)),
                     axis=1, keepdims=True)
        ya = jnp.sum(jnp.where(cols == bi, yc, 0.0), axis=1, keepdims=True)
        stats_ref[c, 0, :] = m[:, 0]
        stats_ref[c, 1, :] = s[:, 0]
        stats_ref[c, 2, :] = t[:, 0]
        stats_ref[c, 3, :] = bv[:, 0]
        stats_ref[c, 4, :] = ya[:, 0]
        stats_ref[c, 5, :] = bi[:, 0].astype(jnp.float32)


def _tc_stats(x, W, b2, g):
    return pl.pallas_call(
        _tc_body,
        in_specs=[
            pl.BlockSpec(memory_space=pltpu.MemorySpace.VMEM),
            pl.BlockSpec(memory_space=pltpu.MemorySpace.VMEM),
            pl.BlockSpec(memory_space=pltpu.MemorySpace.VMEM),
            pl.BlockSpec(memory_space=pl.ANY),
        ],
        out_specs=pl.BlockSpec(memory_space=pltpu.MemorySpace.VMEM),
        out_shape=jax.ShapeDtypeStruct((_NA, 8, _B), jnp.float32),
        scratch_shapes=[
            pltpu.VMEM((_NS, _KBLK, _A), jnp.float32),
            pltpu.SemaphoreType.DMA((_NS,)),
        ],
        compiler_params=pltpu.CompilerParams(
            vmem_limit_bytes=100 * 1024 * 1024,
        ),
    )(x, b2, g, W)


def _sc_ln(v):
    bits = plsc.bitcast(v, jnp.int32)
    e = (bits >> 23) - 127
    mbits = (bits & jnp.int32(0x7FFFFF)) | jnp.int32(0x3F800000)
    m = plsc.bitcast(mbits, jnp.float32)
    big = m > _SQRT2
    m = jnp.where(big, m * 0.5, m)
    e = jnp.where(big, e + 1, e)
    ef = e.astype(jnp.float32)
    r = (m - 1.0) / (m + 1.0)
    r2 = r * r
    at = r * (1.0 + r2 * (1.0 / 3.0 + r2 * (0.2 + r2 * (1.0 / 7.0))))
    return ef * _LN2 + 2.0 * at


def _sc_merge(stats_hbm, act_hbm, nlp_hbm, ent_hbm,
              vin, vact, vnlp, vent):
    nc = 2
    wid = lax.axis_index("s") * nc + lax.axis_index("c")
    nrow_w = 16
    nw = _B // nrow_w

    @pl.when(wid < nw)
    def _work():
        base = wid * nrow_w
        pltpu.sync_copy(stats_hbm, vin)
        sl = pl.ds(base, nrow_w)

        m = vin[0, 0, sl]
        s = vin[0, 1, sl]
        t = vin[0, 2, sl]
        bv = vin[0, 3, sl]
        ya = vin[0, 4, sl]
        bi = vin[0, 5, sl]
        for c in range(1, _NA):
            mc = vin[c, 0, sl]
            sc = vin[c, 1, sl]
            tc = vin[c, 2, sl]
            bvc = vin[c, 3, sl]
            yac = vin[c, 4, sl]
            bic = vin[c, 5, sl]
            mn = jnp.maximum(m, mc)
            a0 = jnp.exp(m - mn)
            a1 = jnp.exp(mc - mn)
            s = s * a0 + sc * a1
            t = t * a0 + tc * a1
            m = mn
            upd = bvc > bv
            bv = jnp.where(upd, bvc, bv)
            bi = jnp.where(upd, bic, bi)
            ya = jnp.where(upd, yac, ya)
        logz = m + _sc_ln(s)
        vact[...] = bi.astype(jnp.int32)
        vnlp[...] = logz - ya
        vent[...] = logz - t / s
        pltpu.sync_copy(vact, act_hbm.at[pl.ds(base, nrow_w)])
        pltpu.sync_copy(vnlp, nlp_hbm.at[pl.ds(base, nrow_w)])
        pltpu.sync_copy(vent, ent_hbm.at[pl.ds(base, nrow_w)])


def _sc_finalize(stats):
    mesh = plsc.VectorSubcoreMesh(core_axis_name="c", subcore_axis_name="s")
    fn = pl.kernel(
        _sc_merge,
        mesh=mesh,
        out_type=[
            jax.ShapeDtypeStruct((_B,), jnp.int32),
            jax.ShapeDtypeStruct((_B,), jnp.float32),
            jax.ShapeDtypeStruct((_B,), jnp.float32),
        ],
        scratch_types=[
            pltpu.VMEM((_NA, 8, _B), jnp.float32),
            pltpu.VMEM((16,), jnp.int32),
            pltpu.VMEM((16,), jnp.float32),
            pltpu.VMEM((16,), jnp.float32),
        ],
        compiler_params=pltpu.CompilerParams(needs_layout_passes=False),
    )
    return fn(stats)


def kernel(x, W, b):
    g = jnp.asarray(_G)
    b2 = b.reshape(1, _A)
    stats = _tc_stats(x, W, b2, g)
    act, nlp, ent = _sc_finalize(stats)
    return (act, nlp, ent)
